# baseline (speedup 1.0000x reference)
import functools

import jax
import jax.numpy as jnp
from jax import lax
from jax.experimental import pallas as pl
from jax.experimental.pallas import tpu as pltpu
from jax.experimental.pallas import tpu_sc as plsc

B = 4096
T = 200
D = 64
N = B * T

NC = 2
NS = 16
NW = NC * NS

ROWS_PER_W = B // NW
CR = 2
CHUNK = CR * T
NCHUNK = ROWS_PER_W // CR
NBUF = 4
NSUPER = NCHUNK // NBUF


def _make_kernel():
    mesh = plsc.VectorSubcoreMesh(core_axis_name="c", subcore_axis_name="s")

    scratch = (
        [pltpu.VMEM((CHUNK,), jnp.int32) for _ in range(NBUF)]
        + [pltpu.VMEM((CHUNK,), jnp.int32) for _ in range(NBUF)]
        + [pltpu.VMEM((CHUNK, D), jnp.float32) for _ in range(NBUF)]
        + [pltpu.VMEM((CHUNK,), jnp.int32)]
        + [pltpu.SemaphoreType.DMA for _ in range(4 * NBUF)]
    )

    @functools.partial(
        pl.kernel,
        out_type=jax.ShapeDtypeStruct((N, D), jnp.float32),
        mesh=mesh,
        scratch_types=scratch,
        compiler_params=pltpu.CompilerParams(use_tc_tiling_on_sc=False),
    )
    def embed_kernel(xt_hbm, yt_hbm, xtab_hbm, ytab_hbm, ptab_hbm, out_hbm,
                     *refs):
        xidx = refs[0:NBUF]
        yidx = refs[NBUF:2 * NBUF]
        bufs = refs[2 * NBUF:3 * NBUF]
        posidx = refs[3 * NBUF]
        sems = refs[3 * NBUF + 1:]
        semA = sems[0:NBUF]
        semB = sems[NBUF:2 * NBUF]
        semC = sems[2 * NBUF:3 * NBUF]
        semD = sems[3 * NBUF:4 * NBUF]

        wid = lax.axis_index("s") * NC + lax.axis_index("c")
        base_tok = wid * (ROWS_PER_W * T)

        for k in range(CHUNK // 16):
            posidx[pl.ds(k * 16, 16)] = (
                lax.iota(jnp.int32, 16) + (k * 16)
            ) % T

        def superstep(g, _):
            descB = []
            for b in range(NBUF):
                @pl.when(g > 0)
                def _(b=b):
                    pltpu.make_async_copy(
                        out_hbm.at[pl.ds(0, CHUNK)], bufs[b], semD[b]
                    ).wait()
                descB.append(
                    pltpu.async_copy(ptab_hbm.at[posidx], bufs[b], semB[b])
                )

            descC = []
            for b in range(NBUF):
                ci = g * NBUF + b
                tok0 = base_tok + ci * CHUNK

                @pl.when(g > 0)
                def _(b=b):
                    pltpu.make_async_copy(
                        xt_hbm.at[pl.ds(0, CHUNK)], xidx[b], semA[b]
                    ).wait()
                    pltpu.make_async_copy(
                        yt_hbm.at[pl.ds(0, CHUNK)], yidx[b], semA[b]
                    ).wait()

                @pl.when(g == 0)
                def _(b=b, tok0=tok0):
                    pltpu.sync_copy(xt_hbm.at[pl.ds(tok0, CHUNK)], xidx[b])
                    pltpu.sync_copy(yt_hbm.at[pl.ds(tok0, CHUNK)], yidx[b])

                descB[b].wait()
                descC.append((
                    pltpu.async_copy(
                        xtab_hbm.at[xidx[b]], bufs[b], semC[b], add=True),
                    pltpu.async_copy(
                        ytab_hbm.at[yidx[b]], bufs[b], semC[b], add=True),
                ))

            for b in range(NBUF):
                ci = g * NBUF + b
                tok0 = base_tok + ci * CHUNK
                descC[b][0].wait()
                descC[b][1].wait()
                pltpu.async_copy(bufs[b], out_hbm.at[pl.ds(tok0, CHUNK)],
                                 semD[b])

                @pl.when(g + 1 < NSUPER)
                def _(b=b, tok0=tok0):
                    tok1 = tok0 + NBUF * CHUNK
                    pltpu.async_copy(
                        xt_hbm.at[pl.ds(tok1, CHUNK)], xidx[b], semA[b])
                    pltpu.async_copy(
                        yt_hbm.at[pl.ds(tok1, CHUNK)], yidx[b], semA[b])
            return ()

        lax.fori_loop(0, NSUPER, superstep, ())

        for b in range(NBUF):
            pltpu.make_async_copy(
                out_hbm.at[pl.ds(0, CHUNK)], bufs[b], semD[b]
            ).wait()

    return embed_kernel


_kernel = _make_kernel()


@jax.jit
def kernel(x_tokens, y_tokens, x_table, y_table, pos_table):
    xt = x_tokens.reshape(N).astype(jnp.int32)
    yt = y_tokens.reshape(N).astype(jnp.int32)
    out = _kernel(xt, yt, x_table, y_table, pos_table)
    return out.reshape(B, T, D)


# speedup vs baseline: 1.7937x; 1.7937x over previous
import functools

import jax
import jax.numpy as jnp
from jax import lax
from jax.experimental import pallas as pl
from jax.experimental.pallas import tpu as pltpu
from jax.experimental.pallas import tpu_sc as plsc

B = 4096
T = 200
D = 64
N = B * T

NC = 2
NS = 16
NW = NC * NS

ROWS_PER_W = B // NW
CR = 2
CHUNK = CR * T
NCHUNK = ROWS_PER_W // CR
NBUF = 4
NSUPER = NCHUNK // NBUF


def _make_kernel():
    mesh = plsc.VectorSubcoreMesh(core_axis_name="c", subcore_axis_name="s")

    scratch = (
        [pltpu.VMEM((CHUNK,), jnp.int32) for _ in range(NBUF)]
        + [pltpu.VMEM((CHUNK,), jnp.int32) for _ in range(NBUF)]
        + [pltpu.VMEM((CHUNK, D), jnp.float32) for _ in range(NBUF)]
        + [pltpu.VMEM((CHUNK,), jnp.int32)]
        + [
            pltpu.VMEM_SHARED((1000, D), jnp.float32),
            pltpu.VMEM_SHARED((201, D), jnp.float32),
            pltpu.VMEM_SHARED((T, D), jnp.float32),
        ]
        + [pltpu.SemaphoreType.DMA for _ in range(4 * NBUF)]
    )

    @functools.partial(
        pl.kernel,
        out_type=jax.ShapeDtypeStruct((N, D), jnp.float32),
        mesh=mesh,
        scratch_types=scratch,
        compiler_params=pltpu.CompilerParams(use_tc_tiling_on_sc=False),
    )
    def embed_kernel(xt_hbm, yt_hbm, xtab_hbm, ytab_hbm, ptab_hbm, out_hbm,
                     *refs):
        xidx = refs[0:NBUF]
        yidx = refs[NBUF:2 * NBUF]
        bufs = refs[2 * NBUF:3 * NBUF]
        posidx = refs[3 * NBUF]
        xtab_sp, ytab_sp, ptab_sp = refs[3 * NBUF + 1:3 * NBUF + 4]
        sems = refs[3 * NBUF + 4:]
        semA = sems[0:NBUF]
        semB = sems[NBUF:2 * NBUF]
        semC = sems[2 * NBUF:3 * NBUF]
        semD = sems[3 * NBUF:4 * NBUF]

        wid = lax.axis_index("s") * NC + lax.axis_index("c")
        base_tok = wid * (ROWS_PER_W * T)

        @pl.when(lax.axis_index("s") == 0)
        def _():
            pltpu.sync_copy(xtab_hbm, xtab_sp)
            pltpu.sync_copy(ytab_hbm, ytab_sp)
            pltpu.sync_copy(ptab_hbm, ptab_sp)

        plsc.subcore_barrier()

        for k in range(CHUNK // 16):
            posidx[pl.ds(k * 16, 16)] = (
                lax.iota(jnp.int32, 16) + (k * 16)
            ) % T

        def superstep(g, _):
            descB = []
            for b in range(NBUF):
                @pl.when(g > 0)
                def _(b=b):
                    pltpu.make_async_copy(
                        out_hbm.at[pl.ds(0, CHUNK)], bufs[b], semD[b]
                    ).wait()
                descB.append(
                    pltpu.async_copy(ptab_sp.at[posidx], bufs[b], semB[b])
                )

            descC = []
            for b in range(NBUF):
                ci = g * NBUF + b
                tok0 = base_tok + ci * CHUNK

                @pl.when(g > 0)
                def _(b=b):
                    pltpu.make_async_copy(
                        xt_hbm.at[pl.ds(0, CHUNK)], xidx[b], semA[b]
                    ).wait()
                    pltpu.make_async_copy(
                        yt_hbm.at[pl.ds(0, CHUNK)], yidx[b], semA[b]
                    ).wait()

                @pl.when(g == 0)
                def _(b=b, tok0=tok0):
                    pltpu.sync_copy(xt_hbm.at[pl.ds(tok0, CHUNK)], xidx[b])
                    pltpu.sync_copy(yt_hbm.at[pl.ds(tok0, CHUNK)], yidx[b])

                descB[b].wait()
                descC.append((
                    pltpu.async_copy(
                        xtab_sp.at[xidx[b]], bufs[b], semC[b], add=True),
                    pltpu.async_copy(
                        ytab_sp.at[yidx[b]], bufs[b], semC[b], add=True),
                ))

            for b in range(NBUF):
                ci = g * NBUF + b
                tok0 = base_tok + ci * CHUNK
                descC[b][0].wait()
                descC[b][1].wait()
                pltpu.async_copy(bufs[b], out_hbm.at[pl.ds(tok0, CHUNK)],
                                 semD[b])

                @pl.when(g + 1 < NSUPER)
                def _(b=b, tok0=tok0):
                    tok1 = tok0 + NBUF * CHUNK
                    pltpu.async_copy(
                        xt_hbm.at[pl.ds(tok1, CHUNK)], xidx[b], semA[b])
                    pltpu.async_copy(
                        yt_hbm.at[pl.ds(tok1, CHUNK)], yidx[b], semA[b])
            return ()

        lax.fori_loop(0, NSUPER, superstep, ())

        for b in range(NBUF):
            pltpu.make_async_copy(
                out_hbm.at[pl.ds(0, CHUNK)], bufs[b], semD[b]
            ).wait()

    return embed_kernel


_kernel = _make_kernel()


@jax.jit
def kernel(x_tokens, y_tokens, x_table, y_table, pos_table):
    xt = x_tokens.reshape(N).astype(jnp.int32)
    yt = y_tokens.reshape(N).astype(jnp.int32)
    out = _kernel(xt, yt, x_table, y_table, pos_table)
    return out.reshape(B, T, D)
